# baseline (speedup 1.0000x reference)
import functools

import jax
import jax.numpy as jnp
from jax import lax
from jax.experimental import pallas as pl
from jax.experimental.pallas import tpu as pltpu
from jax.experimental.pallas import tpu_sc as plsc

N = 160000
MAX_NEI = 8
INPUT = 128
HIDDEN = 128
DEPTH = 3

SEG_PLAN = ((64000, 800), (48000, 800), (32000, 800), (16000, 640))

NC = 2
NS = 16
NW = NC * NS
C = 80
NBUF = 5


def _make_sc_gather(seg):
    es = seg * MAX_NEI
    per_w = es // NW
    nchunk = per_w // C
    assert nchunk % NBUF == 0

    def body_fn(h_hbm, idx_hbm, out_hbm, idx_v, rows_v, *sems):
        gsems, ssems = sems[:NBUF], sems[NBUF:]
        wid = lax.axis_index("s") * NC + lax.axis_index("c")
        base = wid * per_w
        pltpu.sync_copy(idx_hbm.at[wid], idx_v)

        def start_g(ci, b):
            pltpu.async_copy(h_hbm.at[idx_v.at[ci]], rows_v.at[b], gsems[b])

        def wait_g(ci, b):
            pltpu.make_async_copy(h_hbm.at[idx_v.at[ci]], rows_v.at[b],
                                  gsems[b]).wait()

        def start_s(ci, b):
            pltpu.async_copy(rows_v.at[b], out_hbm.at[pl.ds(base + ci * C, C)],
                             ssems[b])

        def wait_s(ci, b):
            pltpu.make_async_copy(rows_v.at[b],
                                  out_hbm.at[pl.ds(base + ci * C, C)],
                                  ssems[b]).wait()

        def step(ci, b, k0):
            cg = ci + NBUF - 1
            gb = (b + NBUF - 1) % NBUF
            if k0:
                if cg >= NBUF:
                    wait_s(cg - NBUF, gb)
                start_g(cg, gb)
            else:
                @pl.when(cg < nchunk)
                def _():
                    wait_s(cg - NBUF, gb)
                    start_g(cg, gb)

            wait_g(ci, b)
            start_s(ci, b)

        for b in range(NBUF - 1):
            start_g(b, b)
        for b in range(NBUF):
            step(b, b, True)

        def body(k, _):
            c0 = NBUF * k
            for b in range(NBUF):
                step(c0 + b, b, False)
            return 0

        lax.fori_loop(1, nchunk // NBUF, body, 0)
        for b in range(NBUF):
            wait_s(nchunk - NBUF + b, b)

    return pl.kernel(
        body_fn,
        out_type=jax.ShapeDtypeStruct((es, HIDDEN), jnp.float32),
        mesh=plsc.VectorSubcoreMesh(core_axis_name="c", subcore_axis_name="s"),
        scratch_types=[
            pltpu.VMEM((nchunk, C), jnp.int32),
            pltpu.VMEM((NBUF, C, HIDDEN), jnp.float32),
        ] + [pltpu.SemaphoreType.DMA] * (2 * NBUF),
    )


_SC_GATHERS = {seg: _make_sc_gather(seg) for seg, _ in SEG_PLAN}


def _tc_gru_body_acc(hacc_ref, *refs, t, off):
    del hacc_ref
    _tc_gru_body(*refs, t=t, off=off)


def _tc_gru_body(x_ref, hnei_ref, wr_ref, ur_ref, urb_ref,
                 wzx_ref, wzh_ref, wzb_ref, whx_ref, whh_ref, whb_ref,
                 out_ref, t, off):
    xt = x_ref[...]
    r1 = jnp.dot(xt, wr_ref[...], preferred_element_type=jnp.float32)
    urb = urb_ref[...].reshape(1, HIDDEN)

    sum_h = jnp.zeros((t, HIDDEN), jnp.float32)
    sum_g = jnp.zeros((t, HIDDEN), jnp.float32)
    for j in range(MAX_NEI):
        hj = hnei_ref[j]
        r2 = jnp.dot(hj, ur_ref[...], preferred_element_type=jnp.float32)
        r = jax.nn.sigmoid(r1 + r2 + urb)
        sum_h = sum_h + hj
        sum_g = sum_g + r * hj

    z = jax.nn.sigmoid(
        jnp.dot(xt, wzx_ref[...], preferred_element_type=jnp.float32)
        + jnp.dot(sum_h, wzh_ref[...], preferred_element_type=jnp.float32)
        + wzb_ref[...].reshape(1, HIDDEN))
    pre_h = jnp.tanh(
        jnp.dot(xt, whx_ref[...], preferred_element_type=jnp.float32)
        + jnp.dot(sum_g, whh_ref[...], preferred_element_type=jnp.float32)
        + whb_ref[...].reshape(1, HIDDEN))
    h_new = (1.0 - z) * sum_h + z * pre_h

    row = (lax.broadcasted_iota(jnp.int32, (t, HIDDEN), 0)
           + (off + pl.program_id(0) * t))
    out_ref[...] = jnp.where(row == 0, 0.0, h_new)


def _tc_gru_seg(off, seg, t, h_acc, x, hnei, weights):
    t0 = off // t
    wspec = pl.BlockSpec((HIDDEN, HIDDEN), lambda i: (0, 0))
    bspec = pl.BlockSpec((HIDDEN,), lambda i: (0,))
    first = h_acc is None
    body = _tc_gru_body if first else _tc_gru_body_acc
    in_specs = [
        pl.BlockSpec((t, INPUT), lambda i: (t0 + i, 0)),
        pl.BlockSpec((MAX_NEI, t, HIDDEN), lambda i: (0, i, 0)),
        wspec, wspec, bspec, wspec, wspec, bspec, wspec, wspec, bspec,
    ]
    args = (x, hnei, *weights)
    if not first:
        in_specs = [pl.BlockSpec(memory_space=pltpu.HBM)] + in_specs
        args = (h_acc,) + args
    return pl.pallas_call(
        functools.partial(body, t=t, off=off),
        grid=(seg // t,),
        in_specs=in_specs,
        out_specs=pl.BlockSpec((t, HIDDEN), lambda i: (t0 + i, 0)),
        out_shape=jax.ShapeDtypeStruct((N, HIDDEN), jnp.float32),
        input_output_aliases={} if first else {0: 0},
    )(*args)


def kernel(h, x, mess_graph, W_z_w, W_z_b, W_r_w, U_r_w, U_r_b, W_h_w, W_h_b):
    wr = W_r_w.T
    ur = U_r_w.T
    wzx = W_z_w[:, :INPUT].T
    wzh = W_z_w[:, INPUT:].T
    whx = W_h_w[:, :INPUT].T
    whh = W_h_w[:, INPUT:].T
    weights = (wr, ur, U_r_b, wzx, wzh, W_z_b, whx, whh, W_h_b)

    offs, idx = [], []
    off = 0
    for seg, _ in SEG_PLAN:
        nchunk = seg * MAX_NEI // NW // C
        idx.append(mess_graph[off:off + seg].T.reshape(NW, nchunk, C))
        offs.append(off)
        off += seg

    for _ in range(DEPTH):
        acc = None
        for (seg, t), off, idx_s in zip(SEG_PLAN, offs, idx):
            flat = _SC_GATHERS[seg](h, idx_s)
            hnei = flat.reshape(MAX_NEI, seg, HIDDEN)
            acc = _tc_gru_seg(off, seg, t, acc, x, hnei, weights)
        h = acc
    return h


# speedup vs baseline: 1.0211x; 1.0211x over previous
import functools

import jax
import jax.numpy as jnp
from jax import lax
from jax.experimental import pallas as pl
from jax.experimental.pallas import tpu as pltpu
from jax.experimental.pallas import tpu_sc as plsc

N = 160000
MAX_NEI = 8
INPUT = 128
HIDDEN = 128
DEPTH = 3

SEG_PLAN = ((16000, 640), (40000, 800), (48000, 800), (40000, 800),
            (16000, 640))

NC = 2
NS = 16
NW = NC * NS
C = 80
NBUF = 5


def _make_sc_gather(seg):
    es = seg * MAX_NEI
    per_w = es // NW
    nchunk = per_w // C
    assert nchunk % NBUF == 0

    def body_fn(h_hbm, idx_hbm, out_hbm, idx_v, rows_v, *sems):
        gsems, ssems = sems[:NBUF], sems[NBUF:]
        wid = lax.axis_index("s") * NC + lax.axis_index("c")
        base = wid * per_w
        pltpu.sync_copy(idx_hbm.at[wid], idx_v)

        def start_g(ci, b):
            pltpu.async_copy(h_hbm.at[idx_v.at[ci]], rows_v.at[b], gsems[b])

        def wait_g(ci, b):
            pltpu.make_async_copy(h_hbm.at[idx_v.at[ci]], rows_v.at[b],
                                  gsems[b]).wait()

        def start_s(ci, b):
            pltpu.async_copy(rows_v.at[b], out_hbm.at[pl.ds(base + ci * C, C)],
                             ssems[b])

        def wait_s(ci, b):
            pltpu.make_async_copy(rows_v.at[b],
                                  out_hbm.at[pl.ds(base + ci * C, C)],
                                  ssems[b]).wait()

        def step(ci, b, k0):
            cg = ci + NBUF - 1
            gb = (b + NBUF - 1) % NBUF
            if k0:
                if cg >= NBUF:
                    wait_s(cg - NBUF, gb)
                start_g(cg, gb)
            else:
                @pl.when(cg < nchunk)
                def _():
                    wait_s(cg - NBUF, gb)
                    start_g(cg, gb)

            wait_g(ci, b)
            start_s(ci, b)

        for b in range(NBUF - 1):
            start_g(b, b)
        for b in range(NBUF):
            step(b, b, True)

        def body(k, _):
            c0 = NBUF * k
            for b in range(NBUF):
                step(c0 + b, b, False)
            return 0

        lax.fori_loop(1, nchunk // NBUF, body, 0)
        for b in range(NBUF):
            wait_s(nchunk - NBUF + b, b)

    return pl.kernel(
        body_fn,
        out_type=jax.ShapeDtypeStruct((es, HIDDEN), jnp.float32),
        mesh=plsc.VectorSubcoreMesh(core_axis_name="c", subcore_axis_name="s"),
        scratch_types=[
            pltpu.VMEM((nchunk, C), jnp.int32),
            pltpu.VMEM((NBUF, C, HIDDEN), jnp.float32),
        ] + [pltpu.SemaphoreType.DMA] * (2 * NBUF),
    )


_SC_GATHERS = {seg: _make_sc_gather(seg) for seg, _ in SEG_PLAN}


def _tc_gru_body_acc(hacc_ref, *refs, t, off):
    del hacc_ref
    _tc_gru_body(*refs, t=t, off=off)


def _tc_gru_body(x_ref, hnei_ref, wr_ref, ur_ref, urb_ref,
                 wzx_ref, wzh_ref, wzb_ref, whx_ref, whh_ref, whb_ref,
                 out_ref, t, off):
    xt = x_ref[...]
    r1 = jnp.dot(xt, wr_ref[...], preferred_element_type=jnp.float32)
    urb = urb_ref[...].reshape(1, HIDDEN)

    sum_h = jnp.zeros((t, HIDDEN), jnp.float32)
    sum_g = jnp.zeros((t, HIDDEN), jnp.float32)
    for j in range(MAX_NEI):
        hj = hnei_ref[j]
        r2 = jnp.dot(hj, ur_ref[...], preferred_element_type=jnp.float32)
        r = jax.nn.sigmoid(r1 + r2 + urb)
        sum_h = sum_h + hj
        sum_g = sum_g + r * hj

    z = jax.nn.sigmoid(
        jnp.dot(xt, wzx_ref[...], preferred_element_type=jnp.float32)
        + jnp.dot(sum_h, wzh_ref[...], preferred_element_type=jnp.float32)
        + wzb_ref[...].reshape(1, HIDDEN))
    pre_h = jnp.tanh(
        jnp.dot(xt, whx_ref[...], preferred_element_type=jnp.float32)
        + jnp.dot(sum_g, whh_ref[...], preferred_element_type=jnp.float32)
        + whb_ref[...].reshape(1, HIDDEN))
    h_new = (1.0 - z) * sum_h + z * pre_h

    row = (lax.broadcasted_iota(jnp.int32, (t, HIDDEN), 0)
           + (off + pl.program_id(0) * t))
    out_ref[...] = jnp.where(row == 0, 0.0, h_new)


def _tc_gru_seg(off, seg, t, h_acc, x, hnei, weights):
    t0 = off // t
    wspec = pl.BlockSpec((HIDDEN, HIDDEN), lambda i: (0, 0))
    bspec = pl.BlockSpec((HIDDEN,), lambda i: (0,))
    first = h_acc is None
    body = _tc_gru_body if first else _tc_gru_body_acc
    in_specs = [
        pl.BlockSpec((t, INPUT), lambda i: (t0 + i, 0)),
        pl.BlockSpec((MAX_NEI, t, HIDDEN), lambda i: (0, i, 0)),
        wspec, wspec, bspec, wspec, wspec, bspec, wspec, wspec, bspec,
    ]
    args = (x, hnei, *weights)
    if not first:
        in_specs = [pl.BlockSpec(memory_space=pltpu.HBM)] + in_specs
        args = (h_acc,) + args
    return pl.pallas_call(
        functools.partial(body, t=t, off=off),
        grid=(seg // t,),
        in_specs=in_specs,
        out_specs=pl.BlockSpec((t, HIDDEN), lambda i: (t0 + i, 0)),
        out_shape=jax.ShapeDtypeStruct((N, HIDDEN), jnp.float32),
        input_output_aliases={} if first else {0: 0},
    )(*args)


def kernel(h, x, mess_graph, W_z_w, W_z_b, W_r_w, U_r_w, U_r_b, W_h_w, W_h_b):
    wr = W_r_w.T
    ur = U_r_w.T
    wzx = W_z_w[:, :INPUT].T
    wzh = W_z_w[:, INPUT:].T
    whx = W_h_w[:, :INPUT].T
    whh = W_h_w[:, INPUT:].T
    weights = (wr, ur, U_r_b, wzx, wzh, W_z_b, whx, whh, W_h_b)

    offs, idx = [], []
    off = 0
    for seg, _ in SEG_PLAN:
        nchunk = seg * MAX_NEI // NW // C
        idx.append(mess_graph[off:off + seg].T.reshape(NW, nchunk, C))
        offs.append(off)
        off += seg

    for _ in range(DEPTH):
        acc = None
        for (seg, t), off, idx_s in zip(SEG_PLAN, offs, idx):
            flat = _SC_GATHERS[seg](h, idx_s)
            hnei = flat.reshape(MAX_NEI, seg, HIDDEN)
            acc = _tc_gru_seg(off, seg, t, acc, x, hnei, weights)
        h = acc
    return h
